# baseline (speedup 1.0000x reference)
import jax
import jax.numpy as jnp
from jax.experimental import pallas as pl
from jax.experimental.pallas import tpu as pltpu

NUM_SEGMENTS = 512
BV = 2048
_NEG_BIG = -1e30


def _agg_kernel(h_ref, batch_ref, w_ref, b_ref, out_ref, m_ref, den_ref, acc_ref):
    i = pl.program_id(0)
    nb = pl.num_programs(0)

    @pl.when(i == 0)
    def _init():
        m_ref[0, 0] = _NEG_BIG
        den_ref[...] = jnp.zeros_like(den_ref)
        acc_ref[...] = jnp.zeros_like(acc_ref)

    h = h_ref[...]
    batch = batch_ref[0]
    scores = jnp.dot(h, w_ref[...], preferred_element_type=jnp.float32)
    scores = scores + b_ref[0, 0]

    seg_ids = jax.lax.broadcasted_iota(jnp.int32, (NUM_SEGMENTS, BV), 0)
    mask_bf = jnp.where(seg_ids == batch, 1.0, 0.0).astype(jnp.bfloat16)

    m_old = m_ref[0, 0]
    m_new = jnp.maximum(m_old, jnp.max(scores))
    scale = jnp.exp(m_old - m_new)

    p = jnp.exp(scores - m_new)

    weighted = jnp.concatenate(
        [(p * h).astype(jnp.bfloat16), p.astype(jnp.bfloat16)], axis=1
    )
    upd = jnp.dot(mask_bf, weighted, preferred_element_type=jnp.float32)
    acc_ref[...] = acc_ref[...] * scale + upd[:, :-1]
    den_ref[...] = den_ref[...] * scale + upd[:, -1:]
    m_ref[0, 0] = m_new

    @pl.when(i == nb - 1)
    def _fini():
        den = den_ref[...]
        out_ref[...] = jnp.where(den > 0.0, acc_ref[...] / den, 0.0)


@jax.jit
def kernel(H, batch, W, b):
    V, D = H.shape
    nb = (V + BV - 1) // BV
    vpad = nb * BV - V
    if vpad:
        H = jnp.concatenate([H, jnp.zeros((vpad, D), H.dtype)], axis=0)
        batch = jnp.concatenate(
            [batch.astype(jnp.int32), jnp.full((vpad,), NUM_SEGMENTS, jnp.int32)]
        )
    batch_r = batch.astype(jnp.int32).reshape(nb, 1, BV)
    b_r = b.reshape(1, 1).astype(jnp.float32)

    out = pl.pallas_call(
        _agg_kernel,
        grid=(nb,),
        in_specs=[
            pl.BlockSpec((BV, D), lambda i: (i, 0)),
            pl.BlockSpec((1, 1, BV), lambda i: (i, 0, 0)),
            pl.BlockSpec((D, 1), lambda i: (0, 0)),
            pl.BlockSpec((1, 1), lambda i: (0, 0)),
        ],
        out_specs=pl.BlockSpec((NUM_SEGMENTS, D), lambda i: (0, 0)),
        out_shape=jax.ShapeDtypeStruct((NUM_SEGMENTS, D), jnp.float32),
        scratch_shapes=[
            pltpu.SMEM((1, 1), jnp.float32),
            pltpu.VMEM((NUM_SEGMENTS, 1), jnp.float32),
            pltpu.VMEM((NUM_SEGMENTS, D), jnp.float32),
        ],
    )(H, batch_r, W, b_r)
    return out


# speedup vs baseline: 1.2063x; 1.2063x over previous
import jax
import jax.numpy as jnp
from jax.experimental import pallas as pl
from jax.experimental.pallas import tpu as pltpu

NUM_SEGMENTS = 512
BV = 2048
_NEG_BIG = -1e30
_CLAMP = 80.0


def _agg_kernel(h_ref, batch_ref, w_ref, b_ref, out_ref, m_ref, den_ref, acc_ref):
    i = pl.program_id(0)
    nb = pl.num_programs(0)

    h_bf = h_ref[...].astype(jnp.bfloat16)
    batch = batch_ref[0]
    scores = jnp.dot(h_bf, w_ref[...], preferred_element_type=jnp.float32)
    scores_row = scores.reshape(1, BV) + b_ref[0, 0]

    @pl.when(i == 0)
    def _init():
        m_ref[0, 0] = jnp.max(scores_row)
        m_ref[0, 1] = 1.0
        den_ref[...] = jnp.zeros_like(den_ref)
        acc_ref[...] = jnp.zeros_like(acc_ref)

    m_prev = m_ref[0, 0]
    scale = m_ref[0, 1]

    p_row = jnp.exp(jnp.minimum(scores_row - m_prev, _CLAMP))

    seg_ids = jax.lax.broadcasted_iota(jnp.int32, (NUM_SEGMENTS, BV), 0)
    P = jnp.where(seg_ids == batch, p_row, 0.0).astype(jnp.bfloat16)

    upd = jnp.dot(P, h_bf, preferred_element_type=jnp.float32)
    ones = jnp.ones((BV, 128), jnp.bfloat16)
    dupd = jnp.dot(P, ones, preferred_element_type=jnp.float32)

    acc_ref[...] = acc_ref[...] * scale + upd
    den_ref[...] = den_ref[...] * scale + dupd[:, :1]

    m_new = jnp.maximum(m_prev, jnp.max(scores_row))
    m_ref[0, 0] = m_new
    m_ref[0, 1] = jnp.exp(m_prev - m_new)

    @pl.when(i == nb - 1)
    def _fini():
        den = den_ref[...]
        out_ref[...] = jnp.where(den > 0.0, acc_ref[...] / den, 0.0)


@jax.jit
def kernel(H, batch, W, b):
    V, D = H.shape
    nb = (V + BV - 1) // BV
    vpad = nb * BV - V
    if vpad:
        H = jnp.concatenate([H, jnp.zeros((vpad, D), H.dtype)], axis=0)
        batch = jnp.concatenate(
            [batch.astype(jnp.int32), jnp.full((vpad,), NUM_SEGMENTS, jnp.int32)]
        )
    batch_r = batch.astype(jnp.int32).reshape(nb, 1, BV)
    b_r = b.reshape(1, 1).astype(jnp.float32)
    w_bf = W.astype(jnp.bfloat16)

    out = pl.pallas_call(
        _agg_kernel,
        grid=(nb,),
        in_specs=[
            pl.BlockSpec((BV, D), lambda i: (i, 0)),
            pl.BlockSpec((1, 1, BV), lambda i: (i, 0, 0)),
            pl.BlockSpec((D, 1), lambda i: (0, 0)),
            pl.BlockSpec((1, 1), lambda i: (0, 0)),
        ],
        out_specs=pl.BlockSpec((NUM_SEGMENTS, D), lambda i: (0, 0)),
        out_shape=jax.ShapeDtypeStruct((NUM_SEGMENTS, D), jnp.float32),
        scratch_shapes=[
            pltpu.SMEM((1, 2), jnp.float32),
            pltpu.VMEM((NUM_SEGMENTS, 1), jnp.float32),
            pltpu.VMEM((NUM_SEGMENTS, D), jnp.float32),
        ],
    )(H, batch_r, W, b_r)
    return out


# speedup vs baseline: 2.2231x; 1.8429x over previous
import jax
import jax.numpy as jnp
from jax.experimental import pallas as pl
from jax.experimental.pallas import tpu as pltpu

NUM_SEGMENTS = 512
BV = 2000
_NEG_BIG = -1e30
_CLAMP = 80.0


def _agg_kernel(h_ref, batch_ref, w_ref, b_ref, out_ref, m_ref, den_ref, acc_ref):
    i = pl.program_id(0)
    nb = pl.num_programs(0)

    h_bf = h_ref[...].astype(jnp.bfloat16)
    batch = batch_ref[0]
    scores = jnp.dot(h_bf, w_ref[...], preferred_element_type=jnp.float32)
    scores_row = scores.reshape(1, BV) + b_ref[0, 0]

    @pl.when(i == 0)
    def _init():
        m_ref[0, 0] = jnp.max(scores_row)
        m_ref[0, 1] = 1.0
        den_ref[...] = jnp.zeros_like(den_ref)
        acc_ref[...] = jnp.zeros_like(acc_ref)

    m_prev = m_ref[0, 0]
    scale = m_ref[0, 1]

    p_row = jnp.exp(jnp.minimum(scores_row - m_prev, _CLAMP))

    seg_ids = jax.lax.broadcasted_iota(jnp.int32, (NUM_SEGMENTS, BV), 0)
    P = jnp.where(seg_ids == batch, p_row, 0.0).astype(jnp.bfloat16)

    upd = jnp.dot(P, h_bf, preferred_element_type=jnp.float32)
    ones = jnp.ones((BV, 128), jnp.bfloat16)
    dupd = jnp.dot(P, ones, preferred_element_type=jnp.float32)

    acc_ref[...] = acc_ref[...] * scale + upd
    den_ref[...] = den_ref[...] * scale + dupd[:, :1]

    m_new = jnp.maximum(m_prev, jnp.max(scores_row))
    m_ref[0, 0] = m_new
    m_ref[0, 1] = jnp.exp(m_prev - m_new)

    @pl.when(i == nb - 1)
    def _fini():
        den = den_ref[...]
        out_ref[...] = jnp.where(den > 0.0, acc_ref[...] / den, 0.0)


@jax.jit
def kernel(H, batch, W, b):
    V, D = H.shape
    nb = (V + BV - 1) // BV
    vpad = nb * BV - V
    if vpad:
        H = jnp.concatenate([H, jnp.zeros((vpad, D), H.dtype)], axis=0)
        batch = jnp.concatenate(
            [batch.astype(jnp.int32), jnp.full((vpad,), NUM_SEGMENTS, jnp.int32)]
        )
    batch_r = batch.astype(jnp.int32).reshape(nb, 1, BV)
    b_r = b.reshape(1, 1).astype(jnp.float32)
    w_bf = W.astype(jnp.bfloat16)

    out = pl.pallas_call(
        _agg_kernel,
        grid=(nb,),
        in_specs=[
            pl.BlockSpec((BV, D), lambda i: (i, 0)),
            pl.BlockSpec((1, 1, BV), lambda i: (i, 0, 0)),
            pl.BlockSpec((D, 1), lambda i: (0, 0)),
            pl.BlockSpec((1, 1), lambda i: (0, 0)),
        ],
        out_specs=pl.BlockSpec((NUM_SEGMENTS, D), lambda i: (0, 0)),
        out_shape=jax.ShapeDtypeStruct((NUM_SEGMENTS, D), jnp.float32),
        scratch_shapes=[
            pltpu.SMEM((1, 2), jnp.float32),
            pltpu.VMEM((NUM_SEGMENTS, 1), jnp.float32),
            pltpu.VMEM((NUM_SEGMENTS, D), jnp.float32),
        ],
    )(H, batch_r, W, b_r)
    return out


# speedup vs baseline: 3.3035x; 1.4860x over previous
import jax
import jax.numpy as jnp
from jax.experimental import pallas as pl
from jax.experimental.pallas import tpu as pltpu

NUM_SEGMENTS = 512
BV = 4000
LSEG = 128
_CLAMP = 80.0


def _agg_kernel(
    h_ref, batch_ref, w_ref, b_ref, bounds_ref, out_ref, m_ref, den_ref, acc_ref
):
    i = pl.program_id(0)
    nb = pl.num_programs(0)

    h_bf = h_ref[...].astype(jnp.bfloat16)
    batch = batch_ref[0]
    scores = jnp.dot(h_bf, w_ref[...], preferred_element_type=jnp.float32)
    scores_row = scores.reshape(1, BV) + b_ref[0, 0]

    @pl.when(i == 0)
    def _init():
        m_ref[0, 0] = jnp.max(scores_row)
        m_ref[0, 1] = 1.0
        den_ref[...] = jnp.zeros_like(den_ref)
        acc_ref[...] = jnp.zeros_like(acc_ref)

    m_prev = m_ref[0, 0]
    scale = m_ref[0, 1]
    lo8 = bounds_ref[i, 0] * 8
    span = bounds_ref[i, 1]

    p_row = jnp.exp(jnp.minimum(scores_row - m_prev, _CLAMP))

    @pl.when(scale < 1.0)
    def _rescale():
        acc_ref[...] = acc_ref[...] * scale
        den_ref[...] = den_ref[...] * scale

    @pl.when(span < LSEG)
    def _local():
        loc = jax.lax.broadcasted_iota(jnp.int32, (LSEG, BV), 0)
        P = jnp.where(loc == batch - lo8, p_row, 0.0).astype(jnp.bfloat16)
        upd = jnp.dot(P, h_bf, preferred_element_type=jnp.float32)
        ones = jnp.ones((BV, 128), jnp.bfloat16)
        dupd = jnp.dot(P, ones, preferred_element_type=jnp.float32)
        acc_ref[pl.ds(lo8, LSEG), :] += upd
        den_ref[pl.ds(lo8, LSEG), :] += dupd[:, :1]

    @pl.when(span >= LSEG)
    def _full():
        seg_ids = jax.lax.broadcasted_iota(jnp.int32, (NUM_SEGMENTS, BV), 0)
        P = jnp.where(seg_ids == batch, p_row, 0.0).astype(jnp.bfloat16)
        upd = jnp.dot(P, h_bf, preferred_element_type=jnp.float32)
        ones = jnp.ones((BV, 128), jnp.bfloat16)
        dupd = jnp.dot(P, ones, preferred_element_type=jnp.float32)
        acc_ref[...] += upd
        den_ref[...] += dupd[:, :1]

    m_new = jnp.maximum(m_prev, jnp.max(scores_row))
    m_ref[0, 0] = m_new
    m_ref[0, 1] = jnp.exp(m_prev - m_new)

    @pl.when(i == nb - 1)
    def _fini():
        den = den_ref[...]
        out_ref[...] = jnp.where(den > 0.0, acc_ref[...] / den, 0.0)


@jax.jit
def kernel(H, batch, W, b):
    V, D = H.shape
    nb = (V + BV - 1) // BV
    vpad = nb * BV - V
    batch = batch.astype(jnp.int32)
    if vpad:
        H = jnp.concatenate([H, jnp.zeros((vpad, D), H.dtype)], axis=0)
        batch = jnp.concatenate(
            [batch, jnp.full((vpad,), NUM_SEGMENTS, jnp.int32)]
        )
    batch_r = batch.reshape(nb, 1, BV)
    b_r = b.reshape(1, 1).astype(jnp.float32)
    w_bf = W.astype(jnp.bfloat16)

    lo8 = jnp.minimum((batch_r[:, 0, 0] // 8) * 8, NUM_SEGMENTS - LSEG)
    span = batch_r[:, 0, -1] - lo8
    bounds = jnp.stack([lo8 // 8, span], axis=1)

    out = pl.pallas_call(
        _agg_kernel,
        grid=(nb,),
        in_specs=[
            pl.BlockSpec((BV, D), lambda i: (i, 0)),
            pl.BlockSpec((1, 1, BV), lambda i: (i, 0, 0)),
            pl.BlockSpec((D, 1), lambda i: (0, 0)),
            pl.BlockSpec((1, 1), lambda i: (0, 0)),
            pl.BlockSpec((nb, 2), lambda i: (0, 0), memory_space=pltpu.SMEM),
        ],
        out_specs=pl.BlockSpec((NUM_SEGMENTS, D), lambda i: (0, 0)),
        out_shape=jax.ShapeDtypeStruct((NUM_SEGMENTS, D), jnp.float32),
        scratch_shapes=[
            pltpu.SMEM((1, 2), jnp.float32),
            pltpu.VMEM((NUM_SEGMENTS, 1), jnp.float32),
            pltpu.VMEM((NUM_SEGMENTS, D), jnp.float32),
        ],
    )(H, batch_r, w_bf, b_r, bounds)
    return out


# speedup vs baseline: 3.4093x; 1.0320x over previous
import jax
import jax.numpy as jnp
from jax.experimental import pallas as pl
from jax.experimental.pallas import tpu as pltpu

NUM_SEGMENTS = 512
BV = 5000
LSEG = 128
_CLAMP = 80.0


def _agg_kernel(
    h_ref, batch_ref, w_ref, b_ref, bounds_ref, out_ref, m_ref, den_ref, acc_ref
):
    i = pl.program_id(0)
    nb = pl.num_programs(0)

    h_bf = h_ref[...].astype(jnp.bfloat16)
    batch = batch_ref[0]
    scores = jnp.dot(h_bf, w_ref[...], preferred_element_type=jnp.float32)
    scores_row = scores.reshape(1, BV) + b_ref[0, 0]

    @pl.when(i == 0)
    def _init():
        m_ref[0, 0] = jnp.max(scores_row)
        m_ref[0, 1] = 1.0
        den_ref[...] = jnp.zeros_like(den_ref)
        acc_ref[...] = jnp.zeros_like(acc_ref)

    m_prev = m_ref[0, 0]
    scale = m_ref[0, 1]
    lo8 = bounds_ref[i, 0] * 8
    span = bounds_ref[i, 1]

    p_row = jnp.exp(jnp.minimum(scores_row - m_prev, _CLAMP))

    @pl.when(scale < 1.0)
    def _rescale():
        acc_ref[...] = acc_ref[...] * scale
        den_ref[...] = den_ref[...] * scale

    @pl.when(span < LSEG)
    def _local():
        loc = jax.lax.broadcasted_iota(jnp.int32, (LSEG, BV), 0)
        P = jnp.where(loc == batch - lo8, p_row, 0.0).astype(jnp.bfloat16)
        upd = jnp.dot(P, h_bf, preferred_element_type=jnp.float32)
        ones = jnp.ones((BV, 128), jnp.bfloat16)
        dupd = jnp.dot(P, ones, preferred_element_type=jnp.float32)
        acc_ref[pl.ds(lo8, LSEG), :] += upd
        den_ref[pl.ds(lo8, LSEG), :] += dupd[:, :1]

    @pl.when(span >= LSEG)
    def _full():
        seg_ids = jax.lax.broadcasted_iota(jnp.int32, (NUM_SEGMENTS, BV), 0)
        P = jnp.where(seg_ids == batch, p_row, 0.0).astype(jnp.bfloat16)
        upd = jnp.dot(P, h_bf, preferred_element_type=jnp.float32)
        ones = jnp.ones((BV, 128), jnp.bfloat16)
        dupd = jnp.dot(P, ones, preferred_element_type=jnp.float32)
        acc_ref[...] += upd
        den_ref[...] += dupd[:, :1]

    m_new = jnp.maximum(m_prev, jnp.max(scores_row))
    m_ref[0, 0] = m_new
    m_ref[0, 1] = jnp.exp(m_prev - m_new)

    @pl.when(i == nb - 1)
    def _fini():
        den = den_ref[...]
        out_ref[...] = jnp.where(den > 0.0, acc_ref[...] / den, 0.0)


@jax.jit
def kernel(H, batch, W, b):
    V, D = H.shape
    nb = (V + BV - 1) // BV
    vpad = nb * BV - V
    batch = batch.astype(jnp.int32)
    if vpad:
        H = jnp.concatenate([H, jnp.zeros((vpad, D), H.dtype)], axis=0)
        batch = jnp.concatenate(
            [batch, jnp.full((vpad,), NUM_SEGMENTS, jnp.int32)]
        )
    batch_r = batch.reshape(nb, 1, BV)
    b_r = b.reshape(1, 1).astype(jnp.float32)
    w_bf = W.astype(jnp.bfloat16)

    lo8 = jnp.minimum((batch_r[:, 0, 0] // 8) * 8, NUM_SEGMENTS - LSEG)
    span = batch_r[:, 0, -1] - lo8
    bounds = jnp.stack([lo8 // 8, span], axis=1)

    out = pl.pallas_call(
        _agg_kernel,
        grid=(nb,),
        in_specs=[
            pl.BlockSpec((BV, D), lambda i: (i, 0)),
            pl.BlockSpec((1, 1, BV), lambda i: (i, 0, 0)),
            pl.BlockSpec((D, 1), lambda i: (0, 0)),
            pl.BlockSpec((1, 1), lambda i: (0, 0)),
            pl.BlockSpec((nb, 2), lambda i: (0, 0), memory_space=pltpu.SMEM),
        ],
        out_specs=pl.BlockSpec((NUM_SEGMENTS, D), lambda i: (0, 0)),
        out_shape=jax.ShapeDtypeStruct((NUM_SEGMENTS, D), jnp.float32),
        scratch_shapes=[
            pltpu.SMEM((1, 2), jnp.float32),
            pltpu.VMEM((NUM_SEGMENTS, 1), jnp.float32),
            pltpu.VMEM((NUM_SEGMENTS, D), jnp.float32),
        ],
    )(H, batch_r, w_bf, b_r, bounds)
    return out
